# Initial kernel scaffold; baseline (speedup 1.0000x reference)
import jax
import jax.numpy as jnp
from jax.experimental import pallas as pl


def kernel(patient, treatment, confounders, corpus_embeddings, params):
    raise NotImplementedError("write your pallas kernel here")


# speedup vs baseline: 3.4589x; 3.4589x over previous
import functools

import jax
import jax.numpy as jnp
from jax import lax
from jax.experimental import pallas as pl
from jax.experimental.pallas import tpu as pltpu
from jax.experimental.pallas import tpu_sc as plsc

TOPK = 8
T_DIM = 2
EMB = 128
C = 128
DT = 2048
NEG = float('-inf')
HI = lax.Precision.HIGHEST


def _patient_embed(patient, pe_W, pe_b):
    B, pin = patient.shape

    def body(x_ref, w_ref, b_ref, o_ref):
        pe = lax.dot_general(x_ref[...].astype(jnp.bfloat16),
                             w_ref[...].astype(jnp.bfloat16),
                             (((1,), (0,)), ((), ())),
                             preferred_element_type=jnp.float32)
        pe = pe + b_ref[...]
        n = jnp.sqrt(jnp.sum(pe * pe, axis=1, keepdims=True))
        o_ref[...] = pe / jnp.maximum(n, 1e-12)

    return pl.pallas_call(
        body,
        out_shape=jax.ShapeDtypeStruct((B, EMB), jnp.float32),
    )(patient, pe_W, pe_b.reshape(1, EMB))


def _similarity(pe, corpus_pad, n_docs):
    B = pe.shape[0]
    npad = corpus_pad.shape[0]
    steps = npad // DT
    bm_per = DT // C

    def body(pe_ref, c_ref, sim_ref, bm_ref):
        d = pl.program_id(0)
        c = c_ref[...]
        ss = jnp.sum(c * c, axis=1, keepdims=True)
        cn = (c / jnp.maximum(jnp.sqrt(ss), 1e-12)).astype(jnp.bfloat16)
        sim = lax.dot_general(pe_ref[...].astype(jnp.bfloat16), cn,
                              (((1,), (1,)), ((), ())),
                              preferred_element_type=jnp.float32)
        sim_ref[...] = sim

        ids = lax.broadcasted_iota(jnp.int32, (1, DT), 1) + d * DT

        def block_maxima(s):
            return jnp.concatenate(
                [jnp.max(s[:, j * C:(j + 1) * C], axis=1).reshape(1, B)
                 for j in range(bm_per)], axis=0)

        @pl.when(d < steps - 1)
        def _():
            bm_ref[...] = block_maxima(sim)

        @pl.when(d == steps - 1)
        def _():
            bm_ref[...] = block_maxima(jnp.where(ids < n_docs, sim, NEG))

    return pl.pallas_call(
        body,
        grid=(steps,),
        in_specs=[
            pl.BlockSpec((B, EMB), lambda d: (0, 0)),
            pl.BlockSpec((DT, EMB), lambda d: (d, 0)),
        ],
        out_specs=[
            pl.BlockSpec((B, DT), lambda d: (0, d)),
            pl.BlockSpec((bm_per, B), lambda d: (d, 0)),
        ],
        out_shape=[
            jax.ShapeDtypeStruct((B, npad), jnp.float32),
            jax.ShapeDtypeStruct((npad // C, B), jnp.float32),
        ],
    )(pe, corpus_pad)


def _top_blocks(bm_t):
    nb, B = bm_t.shape
    QT = 128

    def body(bm_ref, o_ref):
        x = bm_ref[...]
        iota = lax.broadcasted_iota(jnp.int32, (nb, QT), 0)
        rows = []
        for _ in range(TOPK):
            m = jnp.max(x, axis=0, keepdims=True)
            pos = jnp.min(jnp.where(x >= m, iota, nb), axis=0, keepdims=True)
            rows.append(pos)
            x = jnp.where(iota == pos, NEG, x)
        o_ref[...] = jnp.concatenate(rows, axis=0)

    return pl.pallas_call(
        body,
        grid=(B // QT,),
        in_specs=[pl.BlockSpec((nb, QT), lambda q: (0, q))],
        out_specs=pl.BlockSpec((TOPK, QT), lambda q: (0, q)),
        out_shape=jax.ShapeDtypeStruct((TOPK, B), jnp.int32),
    )(bm_t)


def _sc_gather(table, idx, chunk):
    R = idx.shape[0]
    D = table.shape[1]
    info = plsc.get_sparse_core_info()
    nw = info.num_cores * info.num_subcores
    per_w = R // nw
    chunks = per_w // chunk
    mesh = plsc.VectorSubcoreMesh(core_axis_name="c", subcore_axis_name="s")

    @functools.partial(
        pl.kernel,
        out_type=jax.ShapeDtypeStruct((R, D), jnp.float32),
        mesh=mesh,
        scratch_types=[
            pltpu.VMEM((chunk,), jnp.int32),
            pltpu.VMEM((chunk, D), jnp.float32),
            pltpu.SemaphoreType.DMA,
        ],
    )
    def gk(table_hbm, idx_hbm, out_hbm, idx_v, rows_v, sem):
        wid = lax.axis_index("s") * info.num_cores + lax.axis_index("c")
        for j in range(chunks):
            base = wid * per_w + j * chunk
            pltpu.sync_copy(idx_hbm.at[pl.ds(base, chunk)], idx_v)
            pltpu.async_copy(table_hbm.at[idx_v], rows_v, sem).wait()
            pltpu.sync_copy(rows_v, out_hbm.at[pl.ds(base, chunk)])

    return gk(table, idx)


def _final_topk(cand, bid, n_docs):
    B = cand.shape[0]
    W = TOPK * C
    QT = 128

    def body(c_ref, b_ref, s_ref, i_ref):
        x = c_ref[...]
        bid_t = b_ref[...]
        lane = lax.broadcasted_iota(jnp.int32, (QT, C), 1)
        gids = jnp.concatenate(
            [bid_t[:, j:j + 1] * C + lane for j in range(TOPK)], axis=1)
        x = jnp.where(gids < n_docs, x, NEG)
        iota = lax.broadcasted_iota(jnp.int32, (QT, W), 1)
        vcols, pcols = [], []
        for _ in range(TOPK):
            m = jnp.max(x, axis=1, keepdims=True)
            pos = jnp.min(jnp.where(x >= m, iota, W), axis=1, keepdims=True)
            vcols.append(m)
            pcols.append(pos)
            x = jnp.where(iota == pos, NEG, x)
        pos8 = jnp.concatenate(pcols, axis=1)
        sel = pos8 // C
        lane8 = pos8 % C
        bsel = jnp.zeros((QT, TOPK), jnp.int32)
        for j in range(TOPK):
            bsel = bsel + jnp.where(sel == j, bid_t[:, j:j + 1], 0)
        s_ref[...] = jnp.concatenate(vcols, axis=1)
        i_ref[...] = bsel * C + lane8

    return pl.pallas_call(
        body,
        grid=(B // QT,),
        in_specs=[
            pl.BlockSpec((QT, W), lambda q: (q, 0)),
            pl.BlockSpec((QT, TOPK), lambda q: (q, 0)),
        ],
        out_specs=[
            pl.BlockSpec((QT, TOPK), lambda q: (q, 0)),
            pl.BlockSpec((QT, TOPK), lambda q: (q, 0)),
        ],
        out_shape=[
            jax.ShapeDtypeStruct((B, TOPK), jnp.float32),
            jax.ShapeDtypeStruct((B, TOPK), jnp.int32),
        ],
    )(cand, bid)


def _mlp(conf, flat, treat, p):
    B = conf.shape[0]
    QT = 128
    cd = conf.shape[1]
    hid = p['r1_b'].shape[0]
    half = hid // 2
    quarter = hid // 4

    r1a = p['r1_W'][:cd]
    r1b = p['r1_W'][cd:]
    h1a = p['h1_W'][:half]
    h1t = p['h1_W'][half:]

    def dot(a, b):
        return lax.dot_general(a, b, (((1,), (0,)), ((), ())),
                               preferred_element_type=jnp.float32, precision=HI)

    def ln(h, g, b):
        m = jnp.mean(h, axis=1, keepdims=True)
        d = h - m
        v = jnp.mean(d * d, axis=1, keepdims=True)
        return d / jnp.sqrt(v + 1e-5) * g + b

    def body(conf_ref, flat_ref, tr_ref,
             r1a_ref, r1b_ref, r1b_b, ln1g, ln1b, r2w, r2b, ln2g, ln2b,
             r3w, r3b, h1a_ref, h1t_ref, h1b, h2w, h2b, h3w, h3b,
             p1w, p1b, p2w, p2b,
             fo_ref, cf_ref, pr_ref, rep_ref):
        cf_in = conf_ref[...]
        h = dot(cf_in, r1a_ref[...]) + dot(flat_ref[...], r1b_ref[...]) + r1b_b[...]
        h = jnp.maximum(h, 0.0)
        h = ln(h, ln1g[...], ln1b[...])
        h = dot(h, r2w[...]) + r2b[...]
        h = jnp.maximum(h, 0.0)
        h = ln(h, ln2g[...], ln2b[...])
        rep = dot(h, r3w[...]) + r3b[...]
        rep_ref[...] = rep

        base = dot(rep, h1a_ref[...]) + h1b[...]
        h1t_t = h1t_ref[...]
        outs = []
        for t in range(T_DIM):
            ht = jnp.maximum(base + h1t_t[t:t + 1, :], 0.0)
            ht = jnp.maximum(dot(ht, h2w[...]) + h2b[...], 0.0)
            outs.append(dot(ht, h3w[...]) + h3b[...])
        cf = jnp.concatenate(outs, axis=1)
        cf_ref[...] = cf
        tr = tr_ref[...]
        fo_ref[...] = jnp.sum(tr * cf, axis=1, keepdims=True)

        ph = jnp.maximum(dot(cf_in, p1w[...]) + p1b[...], 0.0)
        logits = dot(ph, p2w[...]) + p2b[...]
        m = jnp.max(logits, axis=1, keepdims=True)
        e = jnp.exp(logits - m)
        pr_ref[...] = e / jnp.sum(e, axis=1, keepdims=True)

    def row_spec(arr):
        return pl.BlockSpec(arr.shape, lambda q: tuple(0 for _ in arr.shape))

    weights = [r1a, r1b, p['r1_b'].reshape(1, -1), p['ln1_g'].reshape(1, -1),
               p['ln1_b'].reshape(1, -1), p['r2_W'], p['r2_b'].reshape(1, -1),
               p['ln2_g'].reshape(1, -1), p['ln2_b'].reshape(1, -1),
               p['r3_W'], p['r3_b'].reshape(1, -1), h1a, h1t,
               p['h1_b'].reshape(1, -1), p['h2_W'], p['h2_b'].reshape(1, -1),
               p['h3_W'], p['h3_b'].reshape(1, -1),
               p['p1_W'], p['p1_b'].reshape(1, -1),
               p['p2_W'], p['p2_b'].reshape(1, -1)]

    return pl.pallas_call(
        body,
        grid=(B // QT,),
        in_specs=[
            pl.BlockSpec((QT, cd), lambda q: (q, 0)),
            pl.BlockSpec((QT, TOPK * EMB), lambda q: (q, 0)),
            pl.BlockSpec((QT, T_DIM), lambda q: (q, 0)),
        ] + [row_spec(w) for w in weights],
        out_specs=[
            pl.BlockSpec((QT, 1), lambda q: (q, 0)),
            pl.BlockSpec((QT, T_DIM), lambda q: (q, 0)),
            pl.BlockSpec((QT, T_DIM), lambda q: (q, 0)),
            pl.BlockSpec((QT, half), lambda q: (q, 0)),
        ],
        out_shape=[
            jax.ShapeDtypeStruct((B, 1), jnp.float32),
            jax.ShapeDtypeStruct((B, T_DIM), jnp.float32),
            jax.ShapeDtypeStruct((B, T_DIM), jnp.float32),
            jax.ShapeDtypeStruct((B, half), jnp.float32),
        ],
    )(conf, flat, treat, *weights)


def kernel(patient, treatment, confounders, corpus_embeddings, params):
    B = patient.shape[0]
    n_docs = corpus_embeddings.shape[0]
    npad = ((n_docs + DT - 1) // DT) * DT
    nb = npad // C

    corpus_pad = jnp.pad(corpus_embeddings, ((0, npad - n_docs), (0, 0)))

    pe = _patient_embed(patient, params['pe_W'], params['pe_b'])
    sim, bm_t = _similarity(pe, corpus_pad, n_docs)
    bid = _top_blocks(bm_t).T

    rows = (jnp.arange(B, dtype=jnp.int32)[:, None] * nb + bid).reshape(-1)
    cand = _sc_gather(sim.reshape(B * nb, C), rows, 128)
    scores, indices = _final_topk(cand.reshape(B, TOPK * C), bid, n_docs)

    retrieved = _sc_gather(corpus_embeddings, indices.reshape(-1), 128)
    flat = retrieved.reshape(B, TOPK * EMB)

    fo, cf, prop, rep = _mlp(confounders, flat, treatment, params)
    return (fo, cf.reshape(B, T_DIM, 1), prop, rep, scores, indices)


# speedup vs baseline: 4.1437x; 1.1980x over previous
import functools

import jax
import jax.numpy as jnp
from jax import lax
from jax.experimental import pallas as pl
from jax.experimental.pallas import tpu as pltpu
from jax.experimental.pallas import tpu_sc as plsc

TOPK = 8
T_DIM = 2
EMB = 128
C = 128
DT = 2048
NEG = float('-inf')
HI = lax.Precision.HIGHEST


def _patient_embed(patient, pe_W, pe_b):
    B, pin = patient.shape

    def body(x_ref, w_ref, b_ref, o_ref):
        pe = lax.dot_general(x_ref[...].astype(jnp.bfloat16),
                             w_ref[...].astype(jnp.bfloat16),
                             (((1,), (0,)), ((), ())),
                             preferred_element_type=jnp.float32)
        pe = pe + b_ref[...]
        n = jnp.sqrt(jnp.sum(pe * pe, axis=1, keepdims=True))
        o_ref[...] = pe / jnp.maximum(n, 1e-12)

    return pl.pallas_call(
        body,
        out_shape=jax.ShapeDtypeStruct((B, EMB), jnp.float32),
    )(patient, pe_W, pe_b.reshape(1, EMB))


def _similarity(pe, corpus, npad, n_docs):
    B = pe.shape[0]
    steps = npad // DT
    bm_per = DT // C

    def body(pe_ref, c_ref, sim_ref, bm_ref):
        d = pl.program_id(0)
        c = c_ref[...]
        ss = jnp.sum(c * c, axis=1, keepdims=True)
        cn = (c / jnp.maximum(jnp.sqrt(ss), 1e-12)).astype(jnp.bfloat16)
        sim = lax.dot_general(pe_ref[...].astype(jnp.bfloat16), cn,
                              (((1,), (1,)), ((), ())),
                              preferred_element_type=jnp.float32)

        for j in range(bm_per):
            sim_ref[j * B:(j + 1) * B, :] = sim[:, j * C:(j + 1) * C]

        def write_bm(s):
            for j in range(bm_per):
                bm_ref[j:j + 1, :] = jnp.max(
                    s[:, j * C:(j + 1) * C], axis=1).reshape(1, B)

        @pl.when(d < steps - 1)
        def _():
            write_bm(sim)

        @pl.when(d == steps - 1)
        def _():
            ids = lax.broadcasted_iota(jnp.int32, (1, DT), 1) + d * DT
            write_bm(jnp.where(ids < n_docs, sim, NEG))

    return pl.pallas_call(
        body,
        grid=(steps,),
        in_specs=[
            pl.BlockSpec((B, EMB), lambda d: (0, 0)),
            pl.BlockSpec((DT, EMB), lambda d: (d, 0)),
        ],
        out_specs=[
            pl.BlockSpec((bm_per * B, C), lambda d: (d, 0)),
            pl.BlockSpec((bm_per, B), lambda d: (d, 0)),
        ],
        out_shape=[
            jax.ShapeDtypeStruct((npad // C * B, C), jnp.float32),
            jax.ShapeDtypeStruct((npad // C, B), jnp.float32),
        ],
    )(pe, corpus)


def _top_blocks(bm_t):
    nb, B = bm_t.shape
    QT = 128

    def body(bm_ref, o_ref):
        x = bm_ref[...]
        iota = lax.broadcasted_iota(jnp.int32, (nb, QT), 0)
        rows = []
        for _ in range(TOPK):
            m = jnp.max(x, axis=0, keepdims=True)
            pos = jnp.min(jnp.where(x >= m, iota, nb), axis=0, keepdims=True)
            rows.append(pos)
            x = jnp.where(iota == pos, NEG, x)
        o_ref[...] = jnp.concatenate(rows, axis=0)

    return pl.pallas_call(
        body,
        grid=(B // QT,),
        in_specs=[pl.BlockSpec((nb, QT), lambda q: (0, q))],
        out_specs=pl.BlockSpec((TOPK, QT), lambda q: (0, q)),
        out_shape=jax.ShapeDtypeStruct((TOPK, B), jnp.int32),
    )(bm_t)


def _sc_gather(table, idx, chunk):
    R = idx.shape[0]
    D = table.shape[1]
    info = plsc.get_sparse_core_info()
    nw = info.num_cores * info.num_subcores
    per_w = R // nw
    chunks = per_w // chunk
    mesh = plsc.VectorSubcoreMesh(core_axis_name="c", subcore_axis_name="s")

    @functools.partial(
        pl.kernel,
        out_type=jax.ShapeDtypeStruct((R, D), jnp.float32),
        mesh=mesh,
        scratch_types=[
            pltpu.VMEM((chunk,), jnp.int32),
            pltpu.VMEM((chunk, D), jnp.float32),
            pltpu.SemaphoreType.DMA,
        ],
    )
    def gk(table_hbm, idx_hbm, out_hbm, idx_v, rows_v, sem):
        wid = lax.axis_index("s") * info.num_cores + lax.axis_index("c")
        for j in range(chunks):
            base = wid * per_w + j * chunk
            pltpu.sync_copy(idx_hbm.at[pl.ds(base, chunk)], idx_v)
            pltpu.async_copy(table_hbm.at[idx_v], rows_v, sem).wait()
            pltpu.sync_copy(rows_v, out_hbm.at[pl.ds(base, chunk)])

    return gk(table, idx)


def _final_topk(cand, bid, n_docs):
    B = cand.shape[0]
    W = TOPK * C
    QT = 128

    def body(c_ref, b_ref, s_ref, i_ref):
        x = c_ref[...]
        bid_t = b_ref[...]
        lane = lax.broadcasted_iota(jnp.int32, (QT, C), 1)
        gids = jnp.concatenate(
            [bid_t[:, j:j + 1] * C + lane for j in range(TOPK)], axis=1)
        x = jnp.where(gids < n_docs, x, NEG)
        iota = lax.broadcasted_iota(jnp.int32, (QT, W), 1)
        vcols, pcols = [], []
        for _ in range(TOPK):
            m = jnp.max(x, axis=1, keepdims=True)
            pos = jnp.min(jnp.where(x >= m, iota, W), axis=1, keepdims=True)
            vcols.append(m)
            pcols.append(pos)
            x = jnp.where(iota == pos, NEG, x)
        pos8 = jnp.concatenate(pcols, axis=1)
        sel = pos8 // C
        lane8 = pos8 % C
        bsel = jnp.zeros((QT, TOPK), jnp.int32)
        for j in range(TOPK):
            bsel = bsel + jnp.where(sel == j, bid_t[:, j:j + 1], 0)
        s_ref[...] = jnp.concatenate(vcols, axis=1)
        i_ref[...] = bsel * C + lane8

    return pl.pallas_call(
        body,
        grid=(B // QT,),
        in_specs=[
            pl.BlockSpec((QT, W), lambda q: (q, 0)),
            pl.BlockSpec((QT, TOPK), lambda q: (q, 0)),
        ],
        out_specs=[
            pl.BlockSpec((QT, TOPK), lambda q: (q, 0)),
            pl.BlockSpec((QT, TOPK), lambda q: (q, 0)),
        ],
        out_shape=[
            jax.ShapeDtypeStruct((B, TOPK), jnp.float32),
            jax.ShapeDtypeStruct((B, TOPK), jnp.int32),
        ],
    )(cand, bid)


def _mlp(conf, flat, treat, p):
    B = conf.shape[0]
    QT = 128
    cd = conf.shape[1]
    hid = p['r1_b'].shape[0]
    half = hid // 2
    quarter = hid // 4

    r1a = p['r1_W'][:cd]
    r1b = p['r1_W'][cd:]
    h1a = p['h1_W'][:half]
    h1t = p['h1_W'][half:]

    def dot(a, b):
        return lax.dot_general(a, b, (((1,), (0,)), ((), ())),
                               preferred_element_type=jnp.float32, precision=HI)

    def ln(h, g, b):
        m = jnp.mean(h, axis=1, keepdims=True)
        d = h - m
        v = jnp.mean(d * d, axis=1, keepdims=True)
        return d / jnp.sqrt(v + 1e-5) * g + b

    def body(conf_ref, flat_ref, tr_ref,
             r1a_ref, r1b_ref, r1b_b, ln1g, ln1b, r2w, r2b, ln2g, ln2b,
             r3w, r3b, h1a_ref, h1t_ref, h1b, h2w, h2b, h3w, h3b,
             p1w, p1b, p2w, p2b,
             fo_ref, cf_ref, pr_ref, rep_ref):
        cf_in = conf_ref[...]
        h = dot(cf_in, r1a_ref[...]) + dot(flat_ref[...], r1b_ref[...]) + r1b_b[...]
        h = jnp.maximum(h, 0.0)
        h = ln(h, ln1g[...], ln1b[...])
        h = dot(h, r2w[...]) + r2b[...]
        h = jnp.maximum(h, 0.0)
        h = ln(h, ln2g[...], ln2b[...])
        rep = dot(h, r3w[...]) + r3b[...]
        rep_ref[...] = rep

        base = dot(rep, h1a_ref[...]) + h1b[...]
        h1t_t = h1t_ref[...]
        outs = []
        for t in range(T_DIM):
            ht = jnp.maximum(base + h1t_t[t:t + 1, :], 0.0)
            ht = jnp.maximum(dot(ht, h2w[...]) + h2b[...], 0.0)
            outs.append(dot(ht, h3w[...]) + h3b[...])
        cf = jnp.concatenate(outs, axis=1)
        cf_ref[...] = cf
        tr = tr_ref[...]
        fo_ref[...] = jnp.sum(tr * cf, axis=1, keepdims=True)

        ph = jnp.maximum(dot(cf_in, p1w[...]) + p1b[...], 0.0)
        logits = dot(ph, p2w[...]) + p2b[...]
        m = jnp.max(logits, axis=1, keepdims=True)
        e = jnp.exp(logits - m)
        pr_ref[...] = e / jnp.sum(e, axis=1, keepdims=True)

    def row_spec(arr):
        return pl.BlockSpec(arr.shape, lambda q: tuple(0 for _ in arr.shape))

    weights = [r1a, r1b, p['r1_b'].reshape(1, -1), p['ln1_g'].reshape(1, -1),
               p['ln1_b'].reshape(1, -1), p['r2_W'], p['r2_b'].reshape(1, -1),
               p['ln2_g'].reshape(1, -1), p['ln2_b'].reshape(1, -1),
               p['r3_W'], p['r3_b'].reshape(1, -1), h1a, h1t,
               p['h1_b'].reshape(1, -1), p['h2_W'], p['h2_b'].reshape(1, -1),
               p['h3_W'], p['h3_b'].reshape(1, -1),
               p['p1_W'], p['p1_b'].reshape(1, -1),
               p['p2_W'], p['p2_b'].reshape(1, -1)]

    return pl.pallas_call(
        body,
        grid=(B // QT,),
        in_specs=[
            pl.BlockSpec((QT, cd), lambda q: (q, 0)),
            pl.BlockSpec((QT, TOPK * EMB), lambda q: (q, 0)),
            pl.BlockSpec((QT, T_DIM), lambda q: (q, 0)),
        ] + [row_spec(w) for w in weights],
        out_specs=[
            pl.BlockSpec((QT, 1), lambda q: (q, 0)),
            pl.BlockSpec((QT, T_DIM), lambda q: (q, 0)),
            pl.BlockSpec((QT, T_DIM), lambda q: (q, 0)),
            pl.BlockSpec((QT, half), lambda q: (q, 0)),
        ],
        out_shape=[
            jax.ShapeDtypeStruct((B, 1), jnp.float32),
            jax.ShapeDtypeStruct((B, T_DIM), jnp.float32),
            jax.ShapeDtypeStruct((B, T_DIM), jnp.float32),
            jax.ShapeDtypeStruct((B, half), jnp.float32),
        ],
    )(conf, flat, treat, *weights)


def kernel(patient, treatment, confounders, corpus_embeddings, params):
    B = patient.shape[0]
    n_docs = corpus_embeddings.shape[0]
    npad = ((n_docs + DT - 1) // DT) * DT
    nb = npad // C

    pe = _patient_embed(patient, params['pe_W'], params['pe_b'])
    sim, bm_t = _similarity(pe, corpus_embeddings, npad, n_docs)
    bid = _top_blocks(bm_t).T

    rows = (bid * B + jnp.arange(B, dtype=jnp.int32)[:, None]).reshape(-1)
    cand = _sc_gather(sim, rows, 128)
    scores, indices = _final_topk(cand.reshape(B, TOPK * C), bid, n_docs)

    retrieved = _sc_gather(corpus_embeddings, indices.reshape(-1), 128)
    flat = retrieved.reshape(B, TOPK * EMB)

    fo, cf, prop, rep = _mlp(confounders, flat, treatment, params)
    return (fo, cf.reshape(B, T_DIM, 1), prop, rep, scores, indices)


# speedup vs baseline: 4.3252x; 1.0438x over previous
import functools

import jax
import jax.numpy as jnp
from jax import lax
from jax.experimental import pallas as pl
from jax.experimental.pallas import tpu as pltpu
from jax.experimental.pallas import tpu_sc as plsc

TOPK = 8
T_DIM = 2
EMB = 128
C = 128
DT = 2048
NEG = float('-inf')
HI = lax.Precision.HIGHEST


def _patient_embed(patient, pe_W, pe_b):
    B, pin = patient.shape

    def body(x_ref, w_ref, b_ref, o_ref):
        pe = lax.dot_general(x_ref[...].astype(jnp.bfloat16),
                             w_ref[...].astype(jnp.bfloat16),
                             (((1,), (0,)), ((), ())),
                             preferred_element_type=jnp.float32)
        pe = pe + b_ref[...]
        n = jnp.sqrt(jnp.sum(pe * pe, axis=1, keepdims=True))
        o_ref[...] = pe / jnp.maximum(n, 1e-12)

    return pl.pallas_call(
        body,
        out_shape=jax.ShapeDtypeStruct((B, EMB), jnp.float32),
    )(patient, pe_W, pe_b.reshape(1, EMB))


def _similarity(pe, corpus, npad, n_docs):
    B = pe.shape[0]
    steps = npad // DT
    bm_per = DT // C

    def body(pe_ref, c_ref, sim_ref, bm_ref):
        d = pl.program_id(0)
        c = c_ref[...]
        ss = jnp.sum(c * c, axis=1, keepdims=True)
        cn = (c / jnp.maximum(jnp.sqrt(ss), 1e-12)).astype(jnp.bfloat16)
        sim = lax.dot_general(pe_ref[...].astype(jnp.bfloat16), cn,
                              (((1,), (1,)), ((), ())),
                              preferred_element_type=jnp.float32)

        for j in range(bm_per):
            sim_ref[j * B:(j + 1) * B, :] = sim[:, j * C:(j + 1) * C]

        def write_bm(s):
            for j in range(bm_per):
                bm_ref[j:j + 1, :] = jnp.max(
                    s[:, j * C:(j + 1) * C], axis=1).reshape(1, B)

        @pl.when(d < steps - 1)
        def _():
            write_bm(sim)

        @pl.when(d == steps - 1)
        def _():
            ids = lax.broadcasted_iota(jnp.int32, (1, DT), 1) + d * DT
            write_bm(jnp.where(ids < n_docs, sim, NEG))

    return pl.pallas_call(
        body,
        grid=(steps,),
        in_specs=[
            pl.BlockSpec((B, EMB), lambda d: (0, 0)),
            pl.BlockSpec((DT, EMB), lambda d: (d, 0)),
        ],
        out_specs=[
            pl.BlockSpec((bm_per * B, C), lambda d: (d, 0)),
            pl.BlockSpec((bm_per, B), lambda d: (d, 0)),
        ],
        out_shape=[
            jax.ShapeDtypeStruct((npad // C * B, C), jnp.float32),
            jax.ShapeDtypeStruct((npad // C, B), jnp.float32),
        ],
    )(pe, corpus)


def _top_blocks(bm_t):
    nb, B = bm_t.shape
    QT = 128

    def body(bm_ref, o_ref):
        x = bm_ref[...]
        iota = lax.broadcasted_iota(jnp.int32, (nb, QT), 0)
        rows = []
        for _ in range(TOPK):
            m = jnp.max(x, axis=0, keepdims=True)
            pos = jnp.min(jnp.where(x >= m, iota, nb), axis=0, keepdims=True)
            rows.append(pos)
            x = jnp.where(iota == pos, NEG, x)
        o_ref[...] = jnp.concatenate(rows, axis=0)

    return pl.pallas_call(
        body,
        grid=(B // QT,),
        in_specs=[pl.BlockSpec((nb, QT), lambda q: (0, q))],
        out_specs=pl.BlockSpec((TOPK, QT), lambda q: (0, q)),
        out_shape=jax.ShapeDtypeStruct((TOPK, B), jnp.int32),
    )(bm_t)


def _sc_gather(table, idx, chunk):
    R = idx.shape[0]
    D = table.shape[1]
    info = plsc.get_sparse_core_info()
    nw = info.num_cores * info.num_subcores
    per_w = R // nw
    chunks = per_w // chunk
    mesh = plsc.VectorSubcoreMesh(core_axis_name="c", subcore_axis_name="s")

    @functools.partial(
        pl.kernel,
        out_type=jax.ShapeDtypeStruct((R, D), jnp.float32),
        mesh=mesh,
        scratch_types=[
            pltpu.VMEM((chunk,), jnp.int32),
            pltpu.VMEM((chunk, D), jnp.float32),
            pltpu.SemaphoreType.DMA,
        ],
    )
    def gk(table_hbm, idx_hbm, out_hbm, idx_v, rows_v, sem):
        wid = lax.axis_index("s") * info.num_cores + lax.axis_index("c")
        for j in range(chunks):
            base = wid * per_w + j * chunk
            pltpu.sync_copy(idx_hbm.at[pl.ds(base, chunk)], idx_v)
            pltpu.async_copy(table_hbm.at[idx_v], rows_v, sem).wait()
            pltpu.sync_copy(rows_v, out_hbm.at[pl.ds(base, chunk)])

    return gk(table, idx)


def _final_topk(cand, bid, n_docs):
    B = cand.shape[0]
    W = TOPK * C
    QT = 128

    def body(c_ref, b_ref, s_ref, i_ref):
        x = c_ref[...]
        bid_t = b_ref[...]
        lane = lax.broadcasted_iota(jnp.int32, (QT, C), 1)
        gids = jnp.concatenate(
            [bid_t[:, j:j + 1] * C + lane for j in range(TOPK)], axis=1)
        x = jnp.where(gids < n_docs, x, NEG)
        iota = lax.broadcasted_iota(jnp.int32, (QT, W), 1)
        vcols, pcols = [], []
        for _ in range(TOPK):
            m = jnp.max(x, axis=1, keepdims=True)
            pos = jnp.min(jnp.where(x >= m, iota, W), axis=1, keepdims=True)
            vcols.append(m)
            pcols.append(pos)
            x = jnp.where(iota == pos, NEG, x)
        pos8 = jnp.concatenate(pcols, axis=1)
        sel = pos8 // C
        lane8 = pos8 % C
        bsel = jnp.zeros((QT, TOPK), jnp.int32)
        for j in range(TOPK):
            bsel = bsel + jnp.where(sel == j, bid_t[:, j:j + 1], 0)
        s_ref[...] = jnp.concatenate(vcols, axis=1)
        i_ref[...] = bsel * C + lane8

    return pl.pallas_call(
        body,
        grid=(B // QT,),
        in_specs=[
            pl.BlockSpec((QT, W), lambda q: (q, 0)),
            pl.BlockSpec((QT, TOPK), lambda q: (q, 0)),
        ],
        out_specs=[
            pl.BlockSpec((QT, TOPK), lambda q: (q, 0)),
            pl.BlockSpec((QT, TOPK), lambda q: (q, 0)),
        ],
        out_shape=[
            jax.ShapeDtypeStruct((B, TOPK), jnp.float32),
            jax.ShapeDtypeStruct((B, TOPK), jnp.int32),
        ],
    )(cand, bid)


def _mlp(conf, flat, treat, p):
    B = conf.shape[0]
    QT = 128
    cd = conf.shape[1]
    hid = p['r1_b'].shape[0]
    half = hid // 2
    quarter = hid // 4

    r1a = p['r1_W'][:cd]
    r1b = p['r1_W'][cd:]
    h1a = p['h1_W'][:half]
    h1t = p['h1_W'][half:]

    def dot(a, b):
        return lax.dot_general(a.astype(jnp.bfloat16), b.astype(jnp.bfloat16),
                               (((1,), (0,)), ((), ())),
                               preferred_element_type=jnp.float32)

    def ln(h, g, b):
        m = jnp.mean(h, axis=1, keepdims=True)
        d = h - m
        v = jnp.mean(d * d, axis=1, keepdims=True)
        return d / jnp.sqrt(v + 1e-5) * g + b

    def body(conf_ref, flat_ref, tr_ref,
             r1a_ref, r1b_ref, r1b_b, ln1g, ln1b, r2w, r2b, ln2g, ln2b,
             r3w, r3b, h1a_ref, h1t_ref, h1b, h2w, h2b, h3w, h3b,
             p1w, p1b, p2w, p2b,
             fo_ref, cf_ref, pr_ref, rep_ref):
        cf_in = conf_ref[...]
        h = dot(cf_in, r1a_ref[...]) + dot(flat_ref[...], r1b_ref[...]) + r1b_b[...]
        h = jnp.maximum(h, 0.0)
        h = ln(h, ln1g[...], ln1b[...])
        h = dot(h, r2w[...]) + r2b[...]
        h = jnp.maximum(h, 0.0)
        h = ln(h, ln2g[...], ln2b[...])
        rep = dot(h, r3w[...]) + r3b[...]
        rep_ref[...] = rep

        base = dot(rep, h1a_ref[...]) + h1b[...]
        h1t_t = h1t_ref[...]
        outs = []
        for t in range(T_DIM):
            ht = jnp.maximum(base + h1t_t[t:t + 1, :], 0.0)
            ht = jnp.maximum(dot(ht, h2w[...]) + h2b[...], 0.0)
            outs.append(dot(ht, h3w[...]) + h3b[...])
        cf = jnp.concatenate(outs, axis=1)
        cf_ref[...] = cf
        tr = tr_ref[...]
        fo_ref[...] = jnp.sum(tr * cf, axis=1, keepdims=True)

        ph = jnp.maximum(dot(cf_in, p1w[...]) + p1b[...], 0.0)
        logits = dot(ph, p2w[...]) + p2b[...]
        m = jnp.max(logits, axis=1, keepdims=True)
        e = jnp.exp(logits - m)
        pr_ref[...] = e / jnp.sum(e, axis=1, keepdims=True)

    def row_spec(arr):
        return pl.BlockSpec(arr.shape, lambda q: tuple(0 for _ in arr.shape))

    weights = [r1a, r1b, p['r1_b'].reshape(1, -1), p['ln1_g'].reshape(1, -1),
               p['ln1_b'].reshape(1, -1), p['r2_W'], p['r2_b'].reshape(1, -1),
               p['ln2_g'].reshape(1, -1), p['ln2_b'].reshape(1, -1),
               p['r3_W'], p['r3_b'].reshape(1, -1), h1a, h1t,
               p['h1_b'].reshape(1, -1), p['h2_W'], p['h2_b'].reshape(1, -1),
               p['h3_W'], p['h3_b'].reshape(1, -1),
               p['p1_W'], p['p1_b'].reshape(1, -1),
               p['p2_W'], p['p2_b'].reshape(1, -1)]

    return pl.pallas_call(
        body,
        grid=(B // QT,),
        in_specs=[
            pl.BlockSpec((QT, cd), lambda q: (q, 0)),
            pl.BlockSpec((QT, TOPK * EMB), lambda q: (q, 0)),
            pl.BlockSpec((QT, T_DIM), lambda q: (q, 0)),
        ] + [row_spec(w) for w in weights],
        out_specs=[
            pl.BlockSpec((QT, 1), lambda q: (q, 0)),
            pl.BlockSpec((QT, T_DIM), lambda q: (q, 0)),
            pl.BlockSpec((QT, T_DIM), lambda q: (q, 0)),
            pl.BlockSpec((QT, half), lambda q: (q, 0)),
        ],
        out_shape=[
            jax.ShapeDtypeStruct((B, 1), jnp.float32),
            jax.ShapeDtypeStruct((B, T_DIM), jnp.float32),
            jax.ShapeDtypeStruct((B, T_DIM), jnp.float32),
            jax.ShapeDtypeStruct((B, half), jnp.float32),
        ],
    )(conf, flat, treat, *weights)


def kernel(patient, treatment, confounders, corpus_embeddings, params):
    B = patient.shape[0]
    n_docs = corpus_embeddings.shape[0]
    npad = ((n_docs + DT - 1) // DT) * DT
    nb = npad // C

    pe = _patient_embed(patient, params['pe_W'], params['pe_b'])
    sim, bm_t = _similarity(pe, corpus_embeddings, npad, n_docs)
    bid = _top_blocks(bm_t).T

    rows = (bid * B + jnp.arange(B, dtype=jnp.int32)[:, None]).reshape(-1)
    cand = _sc_gather(sim, rows, 128)
    scores, indices = _final_topk(cand.reshape(B, TOPK * C), bid, n_docs)

    retrieved = _sc_gather(corpus_embeddings, indices.reshape(-1), 128)
    flat = retrieved.reshape(B, TOPK * EMB)

    fo, cf, prop, rep = _mlp(confounders, flat, treatment, params)
    return (fo, cf.reshape(B, T_DIM, 1), prop, rep, scores, indices)


# speedup vs baseline: 6.7208x; 1.5539x over previous
import functools

import jax
import jax.numpy as jnp
from jax import lax
from jax.experimental import pallas as pl
from jax.experimental.pallas import tpu as pltpu
from jax.experimental.pallas import tpu_sc as plsc

TOPK = 8
T_DIM = 2
EMB = 128
C = 128
DT = 2048
NEG = float('-inf')
HI = lax.Precision.HIGHEST


def _patient_embed(patient, pe_W, pe_b):
    B, pin = patient.shape

    def body(x_ref, w_ref, b_ref, o_ref):
        pe = lax.dot_general(x_ref[...].astype(jnp.bfloat16),
                             w_ref[...].astype(jnp.bfloat16),
                             (((1,), (0,)), ((), ())),
                             preferred_element_type=jnp.float32)
        pe = pe + b_ref[...]
        n = jnp.sqrt(jnp.sum(pe * pe, axis=1, keepdims=True))
        o_ref[...] = pe / jnp.maximum(n, 1e-12)

    return pl.pallas_call(
        body,
        out_shape=jax.ShapeDtypeStruct((B, EMB), jnp.float32),
    )(patient, pe_W, pe_b.reshape(1, EMB))


def _similarity(pe, corpus, npad, n_docs):
    B = pe.shape[0]
    steps = npad // DT
    bm_per = DT // C

    def body(pe_ref, c_ref, sim_ref, bm_ref):
        d = pl.program_id(0)
        c = c_ref[...]
        ss = jnp.sum(c * c, axis=1, keepdims=True)
        cn = (c / jnp.maximum(jnp.sqrt(ss), 1e-12)).astype(jnp.bfloat16)
        sim = lax.dot_general(pe_ref[...].astype(jnp.bfloat16), cn,
                              (((1,), (1,)), ((), ())),
                              preferred_element_type=jnp.float32)

        for j in range(bm_per):
            sim_ref[j * B:(j + 1) * B, :] = sim[:, j * C:(j + 1) * C]

        def bm_of(s):
            return jnp.max(s.reshape(B, bm_per, C), axis=2).reshape(1, B, bm_per)

        @pl.when(d < steps - 1)
        def _():
            bm_ref[...] = bm_of(sim)

        @pl.when(d == steps - 1)
        def _():
            ids = lax.broadcasted_iota(jnp.int32, (1, DT), 1) + d * DT
            bm_ref[...] = bm_of(jnp.where(ids < n_docs, sim, NEG))

    return pl.pallas_call(
        body,
        grid=(steps,),
        in_specs=[
            pl.BlockSpec((B, EMB), lambda d: (0, 0)),
            pl.BlockSpec((DT, EMB), lambda d: (d, 0)),
        ],
        out_specs=[
            pl.BlockSpec((bm_per * B, C), lambda d: (d, 0)),
            pl.BlockSpec((1, B, bm_per), lambda d: (d, 0, 0)),
        ],
        out_shape=[
            jax.ShapeDtypeStruct((npad // C * B, C), jnp.float32),
            jax.ShapeDtypeStruct((steps, B, bm_per), jnp.float32),
        ],
    )(pe, corpus)


def _top_blocks(bm):
    B, nb = bm.shape
    QT = 128

    def body(bm_ref, o_ref):
        x = bm_ref[...]
        iota = lax.broadcasted_iota(jnp.int32, (QT, nb), 1)
        cols = []
        for _ in range(TOPK):
            m = jnp.max(x, axis=1, keepdims=True)
            pos = jnp.min(jnp.where(x >= m, iota, nb), axis=1, keepdims=True)
            cols.append(pos)
            x = jnp.where(iota == pos, NEG, x)
        o_ref[...] = jnp.concatenate(cols, axis=1)

    return pl.pallas_call(
        body,
        grid=(B // QT,),
        in_specs=[pl.BlockSpec((QT, nb), lambda q: (q, 0))],
        out_specs=pl.BlockSpec((QT, TOPK), lambda q: (q, 0)),
        out_shape=jax.ShapeDtypeStruct((B, TOPK), jnp.int32),
    )(bm)


def _sc_gather(table, idx, chunk):
    R = idx.shape[0]
    D = table.shape[1]
    info = plsc.get_sparse_core_info()
    nw = info.num_cores * info.num_subcores
    per_w = R // nw
    chunks = per_w // chunk
    mesh = plsc.VectorSubcoreMesh(core_axis_name="c", subcore_axis_name="s")

    @functools.partial(
        pl.kernel,
        out_type=jax.ShapeDtypeStruct((R, D), jnp.float32),
        mesh=mesh,
        scratch_types=[
            pltpu.VMEM((chunk,), jnp.int32),
            pltpu.VMEM((chunk, D), jnp.float32),
            pltpu.SemaphoreType.DMA,
        ],
    )
    def gk(table_hbm, idx_hbm, out_hbm, idx_v, rows_v, sem):
        wid = lax.axis_index("s") * info.num_cores + lax.axis_index("c")
        for j in range(chunks):
            base = wid * per_w + j * chunk
            pltpu.sync_copy(idx_hbm.at[pl.ds(base, chunk)], idx_v)
            pltpu.async_copy(table_hbm.at[idx_v], rows_v, sem).wait()
            pltpu.sync_copy(rows_v, out_hbm.at[pl.ds(base, chunk)])

    return gk(table, idx)


def _final_topk(cand, bid, n_docs):
    B = cand.shape[0]
    W = TOPK * C
    QT = 128

    def body(c_ref, b_ref, s_ref, i_ref):
        x = c_ref[...]
        bid_t = b_ref[...]
        lane = lax.broadcasted_iota(jnp.int32, (QT, C), 1)
        gids = jnp.concatenate(
            [bid_t[:, j:j + 1] * C + lane for j in range(TOPK)], axis=1)
        x = jnp.where(gids < n_docs, x, NEG)
        iota = lax.broadcasted_iota(jnp.int32, (QT, W), 1)
        vcols, pcols = [], []
        for _ in range(TOPK):
            m = jnp.max(x, axis=1, keepdims=True)
            pos = jnp.min(jnp.where(x >= m, iota, W), axis=1, keepdims=True)
            vcols.append(m)
            pcols.append(pos)
            x = jnp.where(iota == pos, NEG, x)
        pos8 = jnp.concatenate(pcols, axis=1)
        sel = pos8 // C
        lane8 = pos8 % C
        bsel = jnp.zeros((QT, TOPK), jnp.int32)
        for j in range(TOPK):
            bsel = bsel + jnp.where(sel == j, bid_t[:, j:j + 1], 0)
        s_ref[...] = jnp.concatenate(vcols, axis=1)
        i_ref[...] = bsel * C + lane8

    return pl.pallas_call(
        body,
        grid=(B // QT,),
        in_specs=[
            pl.BlockSpec((QT, W), lambda q: (q, 0)),
            pl.BlockSpec((QT, TOPK), lambda q: (q, 0)),
        ],
        out_specs=[
            pl.BlockSpec((QT, TOPK), lambda q: (q, 0)),
            pl.BlockSpec((QT, TOPK), lambda q: (q, 0)),
        ],
        out_shape=[
            jax.ShapeDtypeStruct((B, TOPK), jnp.float32),
            jax.ShapeDtypeStruct((B, TOPK), jnp.int32),
        ],
    )(cand, bid)


def _mlp(conf, flat, treat, p):
    B = conf.shape[0]
    QT = 128
    cd = conf.shape[1]
    hid = p['r1_b'].shape[0]
    half = hid // 2
    quarter = hid // 4

    r1a = p['r1_W'][:cd]
    r1b = p['r1_W'][cd:]
    h1a = p['h1_W'][:half]
    h1t = p['h1_W'][half:]

    def dot(a, b):
        return lax.dot_general(a.astype(jnp.bfloat16), b.astype(jnp.bfloat16),
                               (((1,), (0,)), ((), ())),
                               preferred_element_type=jnp.float32)

    def ln(h, g, b):
        m = jnp.mean(h, axis=1, keepdims=True)
        d = h - m
        v = jnp.mean(d * d, axis=1, keepdims=True)
        return d / jnp.sqrt(v + 1e-5) * g + b

    def body(conf_ref, flat_ref, tr_ref,
             r1a_ref, r1b_ref, r1b_b, ln1g, ln1b, r2w, r2b, ln2g, ln2b,
             r3w, r3b, h1a_ref, h1t_ref, h1b, h2w, h2b, h3w, h3b,
             p1w, p1b, p2w, p2b,
             fo_ref, cf_ref, pr_ref, rep_ref):
        cf_in = conf_ref[...]
        h = dot(cf_in, r1a_ref[...]) + dot(flat_ref[...], r1b_ref[...]) + r1b_b[...]
        h = jnp.maximum(h, 0.0)
        h = ln(h, ln1g[...], ln1b[...])
        h = dot(h, r2w[...]) + r2b[...]
        h = jnp.maximum(h, 0.0)
        h = ln(h, ln2g[...], ln2b[...])
        rep = dot(h, r3w[...]) + r3b[...]
        rep_ref[...] = rep

        base = dot(rep, h1a_ref[...]) + h1b[...]
        h1t_t = h1t_ref[...]
        outs = []
        for t in range(T_DIM):
            ht = jnp.maximum(base + h1t_t[t:t + 1, :], 0.0)
            ht = jnp.maximum(dot(ht, h2w[...]) + h2b[...], 0.0)
            outs.append(dot(ht, h3w[...]) + h3b[...])
        cf = jnp.concatenate(outs, axis=1)
        cf_ref[...] = cf
        tr = tr_ref[...]
        fo_ref[...] = jnp.sum(tr * cf, axis=1, keepdims=True)

        ph = jnp.maximum(dot(cf_in, p1w[...]) + p1b[...], 0.0)
        logits = dot(ph, p2w[...]) + p2b[...]
        m = jnp.max(logits, axis=1, keepdims=True)
        e = jnp.exp(logits - m)
        pr_ref[...] = e / jnp.sum(e, axis=1, keepdims=True)

    def row_spec(arr):
        return pl.BlockSpec(arr.shape, lambda q: tuple(0 for _ in arr.shape))

    weights = [r1a, r1b, p['r1_b'].reshape(1, -1), p['ln1_g'].reshape(1, -1),
               p['ln1_b'].reshape(1, -1), p['r2_W'], p['r2_b'].reshape(1, -1),
               p['ln2_g'].reshape(1, -1), p['ln2_b'].reshape(1, -1),
               p['r3_W'], p['r3_b'].reshape(1, -1), h1a, h1t,
               p['h1_b'].reshape(1, -1), p['h2_W'], p['h2_b'].reshape(1, -1),
               p['h3_W'], p['h3_b'].reshape(1, -1),
               p['p1_W'], p['p1_b'].reshape(1, -1),
               p['p2_W'], p['p2_b'].reshape(1, -1)]

    return pl.pallas_call(
        body,
        grid=(B // QT,),
        in_specs=[
            pl.BlockSpec((QT, cd), lambda q: (q, 0)),
            pl.BlockSpec((QT, TOPK * EMB), lambda q: (q, 0)),
            pl.BlockSpec((QT, T_DIM), lambda q: (q, 0)),
        ] + [row_spec(w) for w in weights],
        out_specs=[
            pl.BlockSpec((QT, 1), lambda q: (q, 0)),
            pl.BlockSpec((QT, T_DIM), lambda q: (q, 0)),
            pl.BlockSpec((QT, T_DIM), lambda q: (q, 0)),
            pl.BlockSpec((QT, half), lambda q: (q, 0)),
        ],
        out_shape=[
            jax.ShapeDtypeStruct((B, 1), jnp.float32),
            jax.ShapeDtypeStruct((B, T_DIM), jnp.float32),
            jax.ShapeDtypeStruct((B, T_DIM), jnp.float32),
            jax.ShapeDtypeStruct((B, half), jnp.float32),
        ],
    )(conf, flat, treat, *weights)


def kernel(patient, treatment, confounders, corpus_embeddings, params):
    B = patient.shape[0]
    n_docs = corpus_embeddings.shape[0]
    npad = ((n_docs + DT - 1) // DT) * DT
    nb = npad // C

    pe = _patient_embed(patient, params['pe_W'], params['pe_b'])
    sim, bm3 = _similarity(pe, corpus_embeddings, npad, n_docs)
    bm = bm3.transpose(1, 0, 2).reshape(B, nb)
    bid = _top_blocks(bm)

    rows = (bid * B + jnp.arange(B, dtype=jnp.int32)[:, None]).reshape(-1)
    cand = _sc_gather(sim, rows, 128)
    scores, indices = _final_topk(cand.reshape(B, TOPK * C), bid, n_docs)

    retrieved = _sc_gather(corpus_embeddings, indices.reshape(-1), 128)
    flat = retrieved.reshape(B, TOPK * EMB)

    fo, cf, prop, rep = _mlp(confounders, flat, treatment, params)
    return (fo, cf.reshape(B, T_DIM, 1), prop, rep, scores, indices)


# speedup vs baseline: 7.1697x; 1.0668x over previous
import functools

import jax
import jax.numpy as jnp
from jax import lax
from jax.experimental import pallas as pl
from jax.experimental.pallas import tpu as pltpu
from jax.experimental.pallas import tpu_sc as plsc

TOPK = 8
T_DIM = 2
EMB = 128
C = 128
DT = 2048
NEG = float('-inf')
HI = lax.Precision.HIGHEST


def _similarity(patient, pe_W, pe_b, corpus, npad, n_docs):
    B = patient.shape[0]
    steps = npad // DT
    bm_per = DT // C

    def body(pat_ref, peW_ref, peb_ref, c_ref, sim_ref, bm_ref, pe_scr):
        d = pl.program_id(0)

        @pl.when(d == 0)
        def _():
            pe0 = lax.dot_general(pat_ref[...].astype(jnp.bfloat16),
                                  peW_ref[...].astype(jnp.bfloat16),
                                  (((1,), (0,)), ((), ())),
                                  preferred_element_type=jnp.float32)
            pe0 = pe0 + peb_ref[...]
            n = jnp.sqrt(jnp.sum(pe0 * pe0, axis=1, keepdims=True))
            pe_scr[...] = pe0 / jnp.maximum(n, 1e-12)

        c = c_ref[...]
        ss = jnp.sum(c * c, axis=1, keepdims=True)
        cn = (c / jnp.maximum(jnp.sqrt(ss), 1e-12)).astype(jnp.bfloat16)
        pe_bf = pe_scr[...].astype(jnp.bfloat16)

        def step_work(mask_last):
            cols = []
            for j in range(bm_per):
                sj = lax.dot_general(pe_bf, cn[j * C:(j + 1) * C, :],
                                     (((1,), (1,)), ((), ())),
                                     preferred_element_type=jnp.float32)
                sim_ref[j * B:(j + 1) * B, :] = sj
                if mask_last:
                    ids_j = (lax.broadcasted_iota(jnp.int32, (1, C), 1)
                             + (d * DT + j * C))
                    sj = jnp.where(ids_j < n_docs, sj, NEG)
                cols.append(jnp.max(sj, axis=1, keepdims=True))
            bm_ref[...] = jnp.concatenate(cols, axis=1).reshape(1, B, bm_per)

        @pl.when(d < steps - 1)
        def _():
            step_work(False)

        @pl.when(d == steps - 1)
        def _():
            step_work(True)

    return pl.pallas_call(
        body,
        grid=(steps,),
        in_specs=[
            pl.BlockSpec(patient.shape, lambda d: (0, 0)),
            pl.BlockSpec(pe_W.shape, lambda d: (0, 0)),
            pl.BlockSpec((1, EMB), lambda d: (0, 0)),
            pl.BlockSpec((DT, EMB), lambda d: (d, 0)),
        ],
        out_specs=[
            pl.BlockSpec((bm_per * B, C), lambda d: (d, 0)),
            pl.BlockSpec((1, B, bm_per), lambda d: (d, 0, 0)),
        ],
        out_shape=[
            jax.ShapeDtypeStruct((npad // C * B, C), jnp.float32),
            jax.ShapeDtypeStruct((steps, B, bm_per), jnp.float32),
        ],
        scratch_shapes=[pltpu.VMEM((B, EMB), jnp.float32)],
    )(patient, pe_W, pe_b.reshape(1, EMB), corpus)


def _top_blocks(bm):
    B, nb = bm.shape
    QT = 128

    def body(bm_ref, o_ref):
        x = bm_ref[...]
        iota = lax.broadcasted_iota(jnp.int32, (QT, nb), 1)
        cols = []
        for _ in range(TOPK):
            m = jnp.max(x, axis=1, keepdims=True)
            pos = jnp.min(jnp.where(x >= m, iota, nb), axis=1, keepdims=True)
            cols.append(pos)
            x = jnp.where(iota == pos, NEG, x)
        o_ref[...] = jnp.concatenate(cols, axis=1)

    return pl.pallas_call(
        body,
        grid=(B // QT,),
        in_specs=[pl.BlockSpec((QT, nb), lambda q: (q, 0))],
        out_specs=pl.BlockSpec((QT, TOPK), lambda q: (q, 0)),
        out_shape=jax.ShapeDtypeStruct((B, TOPK), jnp.int32),
    )(bm)


def _sc_gather(table, idx, chunk):
    R = idx.shape[0]
    D = table.shape[1]
    info = plsc.get_sparse_core_info()
    nw = info.num_cores * info.num_subcores
    per_w = R // nw
    chunks = per_w // chunk
    mesh = plsc.VectorSubcoreMesh(core_axis_name="c", subcore_axis_name="s")

    @functools.partial(
        pl.kernel,
        out_type=jax.ShapeDtypeStruct((R, D), jnp.float32),
        mesh=mesh,
        scratch_types=[
            pltpu.VMEM((chunk,), jnp.int32),
            pltpu.VMEM((chunk, D), jnp.float32),
            pltpu.SemaphoreType.DMA,
        ],
    )
    def gk(table_hbm, idx_hbm, out_hbm, idx_v, rows_v, sem):
        wid = lax.axis_index("s") * info.num_cores + lax.axis_index("c")
        for j in range(chunks):
            base = wid * per_w + j * chunk
            pltpu.sync_copy(idx_hbm.at[pl.ds(base, chunk)], idx_v)
            pltpu.async_copy(table_hbm.at[idx_v], rows_v, sem).wait()
            pltpu.sync_copy(rows_v, out_hbm.at[pl.ds(base, chunk)])

    return gk(table, idx)


def _final_topk(cand, bid, n_docs):
    B = cand.shape[0]
    W = TOPK * C
    QT = 128

    def body(c_ref, b_ref, s_ref, i_ref):
        x = c_ref[...]
        bid_t = b_ref[...]
        lane = lax.broadcasted_iota(jnp.int32, (QT, C), 1)
        gids = jnp.concatenate(
            [bid_t[:, j:j + 1] * C + lane for j in range(TOPK)], axis=1)
        x = jnp.where(gids < n_docs, x, NEG)
        iota = lax.broadcasted_iota(jnp.int32, (QT, W), 1)
        vcols, pcols = [], []
        for _ in range(TOPK):
            m = jnp.max(x, axis=1, keepdims=True)
            pos = jnp.min(jnp.where(x >= m, iota, W), axis=1, keepdims=True)
            vcols.append(m)
            pcols.append(pos)
            x = jnp.where(iota == pos, NEG, x)
        pos8 = jnp.concatenate(pcols, axis=1)
        sel = pos8 // C
        lane8 = pos8 % C
        bsel = jnp.zeros((QT, TOPK), jnp.int32)
        for j in range(TOPK):
            bsel = bsel + jnp.where(sel == j, bid_t[:, j:j + 1], 0)
        s_ref[...] = jnp.concatenate(vcols, axis=1)
        i_ref[...] = bsel * C + lane8

    return pl.pallas_call(
        body,
        grid=(B // QT,),
        in_specs=[
            pl.BlockSpec((QT, W), lambda q: (q, 0)),
            pl.BlockSpec((QT, TOPK), lambda q: (q, 0)),
        ],
        out_specs=[
            pl.BlockSpec((QT, TOPK), lambda q: (q, 0)),
            pl.BlockSpec((QT, TOPK), lambda q: (q, 0)),
        ],
        out_shape=[
            jax.ShapeDtypeStruct((B, TOPK), jnp.float32),
            jax.ShapeDtypeStruct((B, TOPK), jnp.int32),
        ],
    )(cand, bid)


def _mlp(conf, flat, treat, p):
    B = conf.shape[0]
    QT = 128
    cd = conf.shape[1]
    hid = p['r1_b'].shape[0]
    half = hid // 2
    quarter = hid // 4

    r1a = p['r1_W'][:cd]
    r1b = p['r1_W'][cd:]
    h1a = p['h1_W'][:half]
    h1t = p['h1_W'][half:]

    def dot(a, b):
        return lax.dot_general(a.astype(jnp.bfloat16), b.astype(jnp.bfloat16),
                               (((1,), (0,)), ((), ())),
                               preferred_element_type=jnp.float32)

    def ln(h, g, b):
        m = jnp.mean(h, axis=1, keepdims=True)
        d = h - m
        v = jnp.mean(d * d, axis=1, keepdims=True)
        return d / jnp.sqrt(v + 1e-5) * g + b

    def body(conf_ref, flat_ref, tr_ref,
             r1a_ref, r1b_ref, r1b_b, ln1g, ln1b, r2w, r2b, ln2g, ln2b,
             r3w, r3b, h1a_ref, h1t_ref, h1b, h2w, h2b, h3w, h3b,
             p1w, p1b, p2w, p2b,
             fo_ref, cf_ref, pr_ref, rep_ref):
        cf_in = conf_ref[...]
        h = dot(cf_in, r1a_ref[...]) + dot(flat_ref[...], r1b_ref[...]) + r1b_b[...]
        h = jnp.maximum(h, 0.0)
        h = ln(h, ln1g[...], ln1b[...])
        h = dot(h, r2w[...]) + r2b[...]
        h = jnp.maximum(h, 0.0)
        h = ln(h, ln2g[...], ln2b[...])
        rep = dot(h, r3w[...]) + r3b[...]
        rep_ref[...] = rep

        base = dot(rep, h1a_ref[...]) + h1b[...]
        h1t_t = h1t_ref[...]
        outs = []
        for t in range(T_DIM):
            ht = jnp.maximum(base + h1t_t[t:t + 1, :], 0.0)
            ht = jnp.maximum(dot(ht, h2w[...]) + h2b[...], 0.0)
            outs.append(dot(ht, h3w[...]) + h3b[...])
        cf = jnp.concatenate(outs, axis=1)
        cf_ref[...] = cf
        tr = tr_ref[...]
        fo_ref[...] = jnp.sum(tr * cf, axis=1, keepdims=True)

        ph = jnp.maximum(dot(cf_in, p1w[...]) + p1b[...], 0.0)
        logits = dot(ph, p2w[...]) + p2b[...]
        m = jnp.max(logits, axis=1, keepdims=True)
        e = jnp.exp(logits - m)
        pr_ref[...] = e / jnp.sum(e, axis=1, keepdims=True)

    def row_spec(arr):
        return pl.BlockSpec(arr.shape, lambda q: tuple(0 for _ in arr.shape))

    weights = [r1a, r1b, p['r1_b'].reshape(1, -1), p['ln1_g'].reshape(1, -1),
               p['ln1_b'].reshape(1, -1), p['r2_W'], p['r2_b'].reshape(1, -1),
               p['ln2_g'].reshape(1, -1), p['ln2_b'].reshape(1, -1),
               p['r3_W'], p['r3_b'].reshape(1, -1), h1a, h1t,
               p['h1_b'].reshape(1, -1), p['h2_W'], p['h2_b'].reshape(1, -1),
               p['h3_W'], p['h3_b'].reshape(1, -1),
               p['p1_W'], p['p1_b'].reshape(1, -1),
               p['p2_W'], p['p2_b'].reshape(1, -1)]

    return pl.pallas_call(
        body,
        grid=(B // QT,),
        in_specs=[
            pl.BlockSpec((QT, cd), lambda q: (q, 0)),
            pl.BlockSpec((QT, TOPK * EMB), lambda q: (q, 0)),
            pl.BlockSpec((QT, T_DIM), lambda q: (q, 0)),
        ] + [row_spec(w) for w in weights],
        out_specs=[
            pl.BlockSpec((QT, 1), lambda q: (q, 0)),
            pl.BlockSpec((QT, T_DIM), lambda q: (q, 0)),
            pl.BlockSpec((QT, T_DIM), lambda q: (q, 0)),
            pl.BlockSpec((QT, half), lambda q: (q, 0)),
        ],
        out_shape=[
            jax.ShapeDtypeStruct((B, 1), jnp.float32),
            jax.ShapeDtypeStruct((B, T_DIM), jnp.float32),
            jax.ShapeDtypeStruct((B, T_DIM), jnp.float32),
            jax.ShapeDtypeStruct((B, half), jnp.float32),
        ],
    )(conf, flat, treat, *weights)


def kernel(patient, treatment, confounders, corpus_embeddings, params):
    B = patient.shape[0]
    n_docs = corpus_embeddings.shape[0]
    npad = ((n_docs + DT - 1) // DT) * DT
    nb = npad // C

    sim, bm3 = _similarity(patient, params['pe_W'], params['pe_b'],
                           corpus_embeddings, npad, n_docs)
    bid = _top_blocks(bm3.transpose(1, 0, 2).reshape(B, nb))

    rows = (bid * B + jnp.arange(B, dtype=jnp.int32)[:, None]).reshape(-1)
    cand = _sc_gather(sim, rows, 128)
    scores, indices = _final_topk(cand.reshape(B, TOPK * C), bid, n_docs)

    retrieved = _sc_gather(corpus_embeddings, indices.reshape(-1), 128)
    flat = retrieved.reshape(B, TOPK * EMB)

    fo, cf, prop, rep = _mlp(confounders, flat, treatment, params)
    return (fo, cf.reshape(B, T_DIM, 1), prop, rep, scores, indices)
